# baseline (speedup 1.0000x reference)
import functools

import jax
import jax.numpy as jnp
from jax.experimental import pallas as pl
from jax.experimental.pallas import tpu as pltpu

_ROWS = 32
_COLS = 1000000
_BLK = 64000
_NBLK = (_COLS + _BLK - 1) // _BLK
_SUB = 8000
_SUBROWS = _BLK // _SUB


def _pass_a_body(x_ref, m_ref, xo_ref, win_ref, val_ref):
    i = pl.program_id(0)

    @pl.when(i == 0)
    def _init():
        val_ref[...] = jnp.full((_ROWS, 1), -jnp.inf, jnp.float32)
        win_ref[...] = jnp.zeros((_ROWS,), jnp.int32)

    xv = x_ref[...]
    xo_ref[...] = xv

    def _merge(valid):
        vm = jnp.where(valid, xv, -jnp.inf)
        bm = jnp.max(vm, axis=1, keepdims=True)
        better = bm > val_ref[...]
        val_ref[...] = jnp.where(better, bm, val_ref[...])
        win_ref[...] = jnp.where(better[:, 0], i, win_ref[...])

    @pl.when(i < _NBLK - 1)
    def _full():
        _merge(m_ref[...] != 0)

    @pl.when(i == _NBLK - 1)
    def _tail():
        lcols = jax.lax.broadcasted_iota(jnp.int32, (_ROWS, _BLK), 1)
        _merge((m_ref[...] != 0) & (lcols < _COLS - (_NBLK - 1) * _BLK))


def _pass_b_body(win_ref, x_ref, m_ref, o_ref):
    r = pl.program_id(0)

    @pl.when(r == 0)
    def _init():
        o_ref[...] = jnp.zeros((_ROWS,), jnp.int32)

    xv = x_ref[0]
    valid = m_ref[0] != 0
    cols = win_ref[r] * _BLK + (
        jax.lax.broadcasted_iota(jnp.int32, (_SUBROWS, _SUB), 0) * _SUB
        + jax.lax.broadcasted_iota(jnp.int32, (_SUBROWS, _SUB), 1)
    )
    in_bounds = cols < _COLS
    vm = jnp.where(valid & in_bounds, xv, -jnp.inf)
    bm = jnp.max(vm)
    big = jnp.int32(2**31 - 1)
    bi = jnp.min(jnp.where(vm == bm, cols, big))
    bi = jnp.where(bm == -jnp.inf, 0, bi)
    rows = jax.lax.broadcasted_iota(jnp.int32, (_ROWS,), 0)
    o_ref[...] = jnp.where(rows == r, bi, o_ref[...])


@functools.partial(jax.jit, static_argnames=("interpret",))
def _masked_argmax(x, mask_u8, interpret=False):
    x_out, winner = pl.pallas_call(
        _pass_a_body,
        grid=(_NBLK,),
        in_specs=[
            pl.BlockSpec((_ROWS, _BLK), lambda i: (0, i)),
            pl.BlockSpec((_ROWS, _BLK), lambda i: (0, i)),
        ],
        out_specs=[
            pl.BlockSpec((_ROWS, _BLK), lambda i: (0, i)),
            pl.BlockSpec((_ROWS,), lambda i: (0,)),
        ],
        out_shape=[
            jax.ShapeDtypeStruct((_ROWS, _COLS), jnp.float32),
            jax.ShapeDtypeStruct((_ROWS,), jnp.int32),
        ],
        scratch_shapes=[
            pltpu.VMEM((_ROWS, 1), jnp.float32),
        ],
        interpret=interpret,
    )(x, mask_u8)

    x3 = x.reshape(_ROWS, _COLS // _SUB, _SUB)
    m3 = mask_u8.reshape(_ROWS, _COLS // _SUB, _SUB)
    idx = pl.pallas_call(
        _pass_b_body,
        grid_spec=pltpu.PrefetchScalarGridSpec(
            num_scalar_prefetch=1,
            grid=(_ROWS,),
            in_specs=[
                pl.BlockSpec((1, _SUBROWS, _SUB), lambda r, w: (r, w[r], 0)),
                pl.BlockSpec((1, _SUBROWS, _SUB), lambda r, w: (r, w[r], 0)),
            ],
            out_specs=pl.BlockSpec((_ROWS,), lambda r, w: (0,)),
        ),
        out_shape=jax.ShapeDtypeStruct((_ROWS,), jnp.int32),
        interpret=interpret,
    )(winner, x3, m3)
    return x_out, idx


def kernel(x, mask):
    m8 = mask.view(jnp.uint8)
    x_out, idx = _masked_argmax(x, m8)
    return (x_out, idx)
